# baseline (speedup 1.0000x reference)
import functools

import jax
import jax.numpy as jnp
from jax import lax
from jax.experimental import pallas as pl
from jax.experimental.pallas import tpu as pltpu
from jax.experimental.pallas import tpu_sc as plsc

B, S, D, F, E, K = 1, 2048, 1024, 2048, 8, 2
CAP = 2048
RT = 256
NT = 24
ZBASE = E * CAP
DBASE = ZBASE + RT
NFILL = 768
NDUMP = E * RT + NFILL
XROWS = DBASE + NDUMP
LIST_N = 2 * S + NDUMP + RT
NC, NS = 2, 16
NW = NC * NS
LPW = LIST_N // NW
CHUNK = 32
NCH = LPW // CHUNK
METW = 128

_INTERPRET = False


def _router_body(x_ref, wg_ref, bg_ref, pairs_ref, pad_ref, tiles_ref):
    x = x_ref[...]
    scores = jnp.dot(x, wg_ref[...], preferred_element_type=jnp.float32)
    scores = scores + bg_ref[...]
    m1 = jnp.max(scores, axis=1, keepdims=True)
    i1 = jnp.argmax(scores, axis=1).astype(jnp.int32)
    col = lax.broadcasted_iota(jnp.int32, (S, E), 1)
    sel1 = col == i1[:, None]
    masked = jnp.where(sel1, -jnp.inf, scores)
    m2 = jnp.max(masked, axis=1, keepdims=True)
    i2 = jnp.argmax(masked, axis=1).astype(jnp.int32)
    sel2 = col == i2[:, None]
    e2v = jnp.exp(m2 - m1)
    den = 1.0 + e2v
    w1v = 1.0 / den
    w2v = e2v / den
    ind1 = jnp.where(sel1, 1.0, 0.0)
    ind2 = jnp.where(sel2, 1.0, 0.0)
    ind = ind1 + ind2

    ri = lax.broadcasted_iota(jnp.int32, (RT, RT), 0)
    ci = lax.broadcasted_iota(jnp.int32, (RT, RT), 1)
    ltri = jnp.where(ri > ci, 1.0, 0.0).astype(jnp.bfloat16)
    base = jnp.zeros((1, E), jnp.float32)
    rows = []
    for c in range(S // RT):
        blk = ind[c * RT:(c + 1) * RT, :]
        part = jnp.dot(ltri, blk.astype(jnp.bfloat16),
                       preferred_element_type=jnp.float32)
        rows.append(part + base)
        base = base + jnp.sum(blk, axis=0, keepdims=True)
    rank_base = jnp.concatenate(rows, axis=0)

    rank1 = jnp.sum(rank_base * ind1, axis=1, keepdims=True)
    rank2 = jnp.sum(rank_base * ind2, axis=1, keepdims=True)
    slot1 = i1[:, None].astype(jnp.float32) * CAP + rank1
    slot2 = i2[:, None].astype(jnp.float32) * CAP + rank2
    pairs_ref[...] = (jnp.where(col == 0, slot1, 0.0)
                      + jnp.where(col == 1, slot2, 0.0)
                      + jnp.where(col == 2, w1v, 0.0)
                      + jnp.where(col == 3, w2v, 0.0))

    ones_col = jnp.full((S, 1), 1.0, jnp.float32)
    cnt_col = lax.dot_general(ind, ones_col, (((0,), (0,)), ((), ())),
                              preferred_element_type=jnp.float32)
    cnti = cnt_col.astype(jnp.int32)
    padcnt = (RT - cnti % RT) % RT
    ntiles = (cnti + padcnt) // RT

    erow = lax.broadcasted_iota(jnp.int32, (E, RT), 0)
    r = lax.broadcasted_iota(jnp.int32, (E, RT), 1)
    pad_ref[...] = jnp.where(r < padcnt, erow * CAP + cnti + r,
                             DBASE + erow * RT + r)

    l8 = jnp.where(lax.broadcasted_iota(jnp.int32, (E, E), 0)
                   > lax.broadcasted_iota(jnp.int32, (E, E), 1), 1.0, 0.0)
    tstart = jnp.dot(l8, ntiles.astype(jnp.float32),
                     preferred_element_type=jnp.float32).astype(jnp.int32)
    tot = jnp.sum(ntiles, axis=0, keepdims=True)
    ti = lax.broadcasted_iota(jnp.int32, (1, 32), 1)
    ge = ti >= tstart
    te = jnp.sum(jnp.where(ge, 1, 0), axis=0, keepdims=True) - 1
    erow32 = lax.broadcasted_iota(jnp.int32, (E, 32), 0)
    onehot = erow32 == te
    tstart_sel = jnp.sum(jnp.where(onehot, tstart, 0), axis=0, keepdims=True)
    ntiles_sel = jnp.sum(jnp.where(onehot, ntiles, 0), axis=0, keepdims=True)
    rb = ti - tstart_sel
    active = ti < tot
    xsblk = jnp.where(active, te * (CAP // RT) + rb, ZBASE // RT)
    te_o = jnp.where(active, te, E - 1)
    first = jnp.where(active & (rb == 0), 1, 0)
    flush = jnp.where(active & (rb == ntiles_sel - 1), 1, 0)
    tiles_ref[0:1, :] = xsblk
    tiles_ref[1:2, :] = te_o
    tiles_ref[2:3, :] = first
    tiles_ref[3:4, :] = flush
    tiles_ref[4:8, :] = jnp.zeros((4, 32), jnp.int32)


def _run_router(x, Wg, bg):
    return pl.pallas_call(
        _router_body,
        grid=(1,),
        in_specs=[
            pl.BlockSpec((S, D), lambda i: (0, 0)),
            pl.BlockSpec((D, E), lambda i: (0, 0)),
            pl.BlockSpec((1, E), lambda i: (0, 0)),
        ],
        out_specs=[
            pl.BlockSpec((S, E), lambda i: (0, 0)),
            pl.BlockSpec((E, RT), lambda i: (0, 0)),
            pl.BlockSpec((8, 32), lambda i: (0, 0)),
        ],
        out_shape=[
            jax.ShapeDtypeStruct((S, E), jnp.float32),
            jax.ShapeDtypeStruct((E, RT), jnp.int32),
            jax.ShapeDtypeStruct((8, 32), jnp.int32),
        ],
        interpret=_INTERPRET,
    )(x, Wg, bg.reshape(1, E))


def _sc_dispatch_body(x_hbm, tok_hbm, slot_hbm, w_hbm, xs_hbm, meta_hbm,
                      tokv, slotv, wv, rows0, rows1, rows2, metal0, metal1,
                      gsems, ssems, msems):
    wid = lax.axis_index("c") * NS + lax.axis_index("s")
    rows = (rows0, rows1, rows2)
    metal = (metal0, metal1)
    lane = lax.broadcasted_iota(jnp.int32, (16,), 0)

    pltpu.sync_copy(tok_hbm.at[wid], tokv)
    pltpu.sync_copy(slot_hbm.at[wid], slotv)
    pltpu.sync_copy(w_hbm.at[wid], wv)

    def start_gather(c):
        return pltpu.async_copy(x_hbm.at[tokv.at[c]], rows[c % 3],
                                gsems[c % 3])

    gh = {}
    sh = {}
    mh = {}
    gh[0] = start_gather(0)
    gh[1] = start_gather(1)
    for c in range(NCH):
        b = c % 3
        mb = c % 2
        if c + 2 < NCH:
            if c >= 1:
                sh.pop(c - 1).wait()
            gh[c + 2] = start_gather(c + 2)
        gh.pop(c).wait()
        if c >= 2:
            mh.pop(c - 2).wait()
        for h in range(2):
            wvec = wv[c, pl.ds(h * 16, 16)]
            for j in range(16):
                metal[mb][h * 16 + j, pl.ds(0, 16)] = jnp.where(
                    lane == 0, wvec[j], 0.0)
        sh[c] = pltpu.async_copy(rows[b], xs_hbm.at[slotv.at[c]], ssems[b])
        mh[c] = pltpu.async_copy(metal[mb], meta_hbm.at[slotv.at[c]],
                                 msems[mb])
    for c in sorted(sh):
        sh[c].wait()
    for c in sorted(mh):
        mh[c].wait()


def _run_sc_dispatch(x, token_list, slot_list, w_list):
    mesh = plsc.VectorSubcoreMesh(core_axis_name="c", subcore_axis_name="s",
                                  num_cores=NC, num_subcores=NS)
    fn = pl.kernel(
        _sc_dispatch_body,
        out_type=[
            jax.ShapeDtypeStruct((XROWS, D), jnp.float32),
            jax.ShapeDtypeStruct((XROWS, METW), jnp.float32),
        ],
        mesh=mesh,
        scratch_types=[
            pltpu.VMEM((NCH, CHUNK), jnp.int32),
            pltpu.VMEM((NCH, CHUNK), jnp.int32),
            pltpu.VMEM((NCH, CHUNK), jnp.float32),
            pltpu.VMEM((CHUNK, D), jnp.float32),
            pltpu.VMEM((CHUNK, D), jnp.float32),
            pltpu.VMEM((CHUNK, D), jnp.float32),
            pltpu.VMEM((CHUNK, METW), jnp.float32),
            pltpu.VMEM((CHUNK, METW), jnp.float32),
            [pltpu.SemaphoreType.DMA] * 3,
            [pltpu.SemaphoreType.DMA] * 3,
            [pltpu.SemaphoreType.DMA] * 2,
        ],
        interpret=_INTERPRET,
    )
    return fn(x, token_list.reshape(NW, NCH, CHUNK),
              slot_list.reshape(NW, NCH, CHUNK),
              w_list.reshape(NW, NCH, CHUNK))


def _expert_body(xsblk_ref, te_ref, first_ref, flush_ref,
                 xs_ref, meta_ref, w1_ref, b1_ref, w2_ref, b2_ref,
                 out_ref, vacc_ref, oacc_ref, tw_ref, wsum_ref):
    i = pl.program_id(0)

    @pl.when(i == 0)
    def _():
        oacc_ref[...] = jnp.zeros_like(oacc_ref)
        tw_ref[0] = 0.0

    @pl.when(first_ref[i] == 1)
    def _():
        vacc_ref[...] = jnp.zeros_like(vacc_ref)
        wsum_ref[0] = 0.0

    h = jnp.dot(xs_ref[...], w1_ref[0], preferred_element_type=jnp.float32)
    h = jnp.maximum(h + b1_ref[0], 0.0)
    wcol = meta_ref[:, 0:1]
    vacc_ref[...] += lax.dot_general(wcol, h, (((0,), (0,)), ((), ())),
                                     preferred_element_type=jnp.float32)
    sw = jnp.sum(wcol)
    wsum_ref[0] += sw
    tw_ref[0] += sw

    @pl.when(flush_ref[i] == 1)
    def _():
        contrib = jnp.dot(vacc_ref[...], w2_ref[0],
                          preferred_element_type=jnp.float32)
        oacc_ref[...] += contrib + wsum_ref[0] * b2_ref[0]

    @pl.when(i == NT - 1)
    def _():
        out_ref[...] = oacc_ref[...] / tw_ref[0]


def _run_experts(xsblk, te, first, flush, xs, meta, W1, b1, W2, b2):
    grid_spec = pltpu.PrefetchScalarGridSpec(
        num_scalar_prefetch=4,
        grid=(NT,),
        in_specs=[
            pl.BlockSpec((RT, D), lambda i, xb, t, fi, fl: (xb[i], 0)),
            pl.BlockSpec((RT, METW), lambda i, xb, t, fi, fl: (xb[i], 0)),
            pl.BlockSpec((1, D, F), lambda i, xb, t, fi, fl: (t[i], 0, 0)),
            pl.BlockSpec((1, 1, F), lambda i, xb, t, fi, fl: (t[i], 0, 0)),
            pl.BlockSpec((1, F, D), lambda i, xb, t, fi, fl: (t[i], 0, 0)),
            pl.BlockSpec((1, 1, D), lambda i, xb, t, fi, fl: (t[i], 0, 0)),
        ],
        out_specs=pl.BlockSpec((1, D), lambda i, xb, t, fi, fl: (0, 0)),
        scratch_shapes=[
            pltpu.VMEM((1, F), jnp.float32),
            pltpu.VMEM((1, D), jnp.float32),
            pltpu.SMEM((1,), jnp.float32),
            pltpu.SMEM((1,), jnp.float32),
        ],
    )
    return pl.pallas_call(
        _expert_body,
        grid_spec=grid_spec,
        out_shape=jax.ShapeDtypeStruct((1, D), jnp.float32),
        compiler_params=pltpu.CompilerParams(
            dimension_semantics=("arbitrary",),
        ),
        interpret=_INTERPRET,
    )(xsblk, te, first, flush, xs, meta, W1, b1.reshape(E, 1, F), W2,
      b2.reshape(E, 1, D))


def kernel(input_tensor, Wg, bg, W1, b1, W2, b2):
    x = input_tensor.reshape(S, D)
    pairs, pad, tiles = _run_router(x, Wg, bg)
    slot1 = pairs[:, 0].astype(jnp.int32)
    slot2 = pairs[:, 1].astype(jnp.int32)
    tok = jnp.arange(S, dtype=jnp.int32)
    token_list = jnp.concatenate([tok, tok,
                                  jnp.zeros((NDUMP + RT,), jnp.int32)])
    slot_list = jnp.concatenate([
        slot1, slot2, pad.reshape(-1),
        ZBASE + jnp.arange(RT, dtype=jnp.int32),
        DBASE + E * RT + jnp.arange(NFILL, dtype=jnp.int32)])
    w_list = jnp.concatenate([pairs[:, 2], pairs[:, 3],
                              jnp.zeros((NDUMP + RT,), jnp.float32)])
    xs, meta = _run_sc_dispatch(x, token_list, slot_list, w_list)
    out = _run_experts(tiles[0], tiles[1], tiles[2], tiles[3],
                       xs, meta, W1, b1, W2, b2)
    return out.reshape(1, 1, D)


# speedup vs baseline: 1.9318x; 1.9318x over previous
import jax
import jax.numpy as jnp
from jax.experimental import pallas as pl
from jax.experimental.pallas import tpu as pltpu

B, S, D, F, E, K = 1, 2048, 1024, 2048, 8, 2
ST = 512
T = S // ST
FC = 512
NFC = F // FC

_INTERPRET = False


def _moe_body(x_ref, wg_ref, bg_ref, w1_ref, b1_ref, w2_ref, b2_ref,
              out_ref, wmask_ref, vacc_ref, oacc_ref, tw_ref, wsum_ref):
    e = pl.program_id(0)
    t = pl.program_id(1)
    x = x_ref[...]

    @pl.when(e == 0)
    def _router():
        scores = jnp.dot(x, wg_ref[...], preferred_element_type=jnp.float32)
        scores = scores + bg_ref[...]
        m1 = jnp.max(scores, axis=1, keepdims=True)
        i1 = jnp.argmax(scores, axis=1).astype(jnp.int32)
        col = jax.lax.broadcasted_iota(jnp.int32, scores.shape, 1)
        sel1 = col == i1[:, None]
        masked = jnp.where(sel1, -jnp.inf, scores)
        m2 = jnp.max(masked, axis=1, keepdims=True)
        i2 = jnp.argmax(masked, axis=1).astype(jnp.int32)
        sel2 = col == i2[:, None]
        e2 = jnp.exp(m2 - m1)
        denom = 1.0 + e2
        wm = jnp.where(sel1, 1.0 / denom, 0.0) + jnp.where(sel2, e2 / denom, 0.0)
        wmask_ref[:, pl.ds(t * ST, ST)] = wm.T
        @pl.when(t == 0)
        def _():
            tw_ref[0] = 0.0
        tw_ref[0] += jnp.sum(wm)

    @pl.when(t == 0)
    def _():
        vacc_ref[...] = jnp.zeros_like(vacc_ref)
        wsum_ref[0] = 0.0

    w_row = wmask_ref[pl.ds(e, 1), pl.ds(t * ST, ST)]
    for fc in range(NFC):
        h = jnp.dot(x, w1_ref[0, :, pl.ds(fc * FC, FC)],
                    preferred_element_type=jnp.float32)
        h = jnp.maximum(h + b1_ref[0, :, pl.ds(fc * FC, FC)], 0.0)
        vacc_ref[0:1, pl.ds(fc * FC, FC)] += jnp.dot(
            w_row, h, preferred_element_type=jnp.float32)
    wsum_ref[0] += jnp.sum(w_row)

    @pl.when(t == T - 1)
    def _finish_expert():
        contrib = jnp.dot(vacc_ref[...], w2_ref[0],
                          preferred_element_type=jnp.float32)
        contrib = contrib + wsum_ref[0] * b2_ref[0]
        @pl.when(e == 0)
        def _():
            oacc_ref[...] = jnp.zeros_like(oacc_ref)
        oacc_ref[...] += contrib
        @pl.when(e == E - 1)
        def _():
            out_ref[...] = oacc_ref[...] / tw_ref[0]


def kernel(input_tensor, Wg, bg, W1, b1, W2, b2):
    x = input_tensor.reshape(S, D)
    out = pl.pallas_call(
        _moe_body,
        grid=(E, T),
        in_specs=[
            pl.BlockSpec((ST, D), lambda e, t: (t, 0)),
            pl.BlockSpec((D, E), lambda e, t: (0, 0)),
            pl.BlockSpec((1, E), lambda e, t: (0, 0)),
            pl.BlockSpec((1, D, F), lambda e, t: (e, 0, 0)),
            pl.BlockSpec((1, 1, F), lambda e, t: (e, 0, 0)),
            pl.BlockSpec((1, F, D), lambda e, t: (e, 0, 0)),
            pl.BlockSpec((1, 1, D), lambda e, t: (e, 0, 0)),
        ],
        out_specs=pl.BlockSpec((1, D), lambda e, t: (0, 0)),
        out_shape=jax.ShapeDtypeStruct((1, D), jnp.float32),
        scratch_shapes=[
            pltpu.VMEM((E, S), jnp.float32),
            pltpu.VMEM((1, F), jnp.float32),
            pltpu.VMEM((1, D), jnp.float32),
            pltpu.SMEM((1,), jnp.float32),
            pltpu.SMEM((1,), jnp.float32),
        ],
        compiler_params=pltpu.CompilerParams(
            dimension_semantics=("arbitrary", "arbitrary"),
        ),
        interpret=_INTERPRET,
    )(x, Wg, bg.reshape(1, E), W1, b1.reshape(E, 1, F), W2, b2.reshape(E, 1, D))
    return out.reshape(1, 1, D)


# speedup vs baseline: 2.4678x; 1.2775x over previous
import jax
import jax.numpy as jnp
from jax.experimental import pallas as pl
from jax.experimental.pallas import tpu as pltpu

B, S, D, F, E, K = 1, 2048, 1024, 2048, 8, 2
ST = 1024
T = S // ST
_INTERPRET = False


def _moe_body(x_ref, wg_ref, bg_ref, w1_ref, b1_ref, w2_ref, b2_ref,
              out_ref, wmask_ref, vacc_ref, oacc_ref, tw_ref, wsum_ref):
    e = pl.program_id(0)
    t = pl.program_id(1)
    x = x_ref[...]

    @pl.when(e == 0)
    def _router():
        scores = jnp.dot(x, wg_ref[...], preferred_element_type=jnp.float32)
        scores = scores + bg_ref[...]
        m1 = jnp.max(scores, axis=1, keepdims=True)
        i1 = jnp.argmax(scores, axis=1).astype(jnp.int32)
        col = jax.lax.broadcasted_iota(jnp.int32, scores.shape, 1)
        sel1 = col == i1[:, None]
        masked = jnp.where(sel1, -jnp.inf, scores)
        m2 = jnp.max(masked, axis=1, keepdims=True)
        i2 = jnp.argmax(masked, axis=1).astype(jnp.int32)
        sel2 = col == i2[:, None]
        e2 = jnp.exp(m2 - m1)
        denom = 1.0 + e2
        wm = jnp.where(sel1, 1.0 / denom, 0.0) + jnp.where(sel2, e2 / denom, 0.0)
        wmask_ref[:, pl.ds(t * ST, ST)] = wm.T
        @pl.when(t == 0)
        def _():
            tw_ref[0] = 0.0
        tw_ref[0] += jnp.sum(wm)

    @pl.when(t == 0)
    def _():
        vacc_ref[...] = jnp.zeros_like(vacc_ref)
        wsum_ref[0] = 0.0

    w_row = wmask_ref[pl.ds(e, 1), pl.ds(t * ST, ST)]
    h = jnp.dot(x, w1_ref[0], preferred_element_type=jnp.float32)
    h = jnp.maximum(h + b1_ref[0], 0.0)
    vacc_ref[...] += jnp.dot(w_row, h, preferred_element_type=jnp.float32)
    wsum_ref[0] += jnp.sum(w_row)

    @pl.when(t == T - 1)
    def _finish_expert():
        contrib = jnp.dot(vacc_ref[...], w2_ref[0],
                          preferred_element_type=jnp.float32)
        contrib = contrib + wsum_ref[0] * b2_ref[0]
        @pl.when(e == 0)
        def _():
            oacc_ref[...] = jnp.zeros_like(oacc_ref)
        oacc_ref[...] += contrib
        @pl.when(e == E - 1)
        def _():
            out_ref[...] = oacc_ref[...] / tw_ref[0]


def kernel(input_tensor, Wg, bg, W1, b1, W2, b2):
    x = input_tensor.reshape(S, D)
    out = pl.pallas_call(
        _moe_body,
        grid=(E, T),
        in_specs=[
            pl.BlockSpec((ST, D), lambda e, t: (t, 0)),
            pl.BlockSpec((D, E), lambda e, t: (0, 0)),
            pl.BlockSpec((1, E), lambda e, t: (0, 0)),
            pl.BlockSpec((1, D, F), lambda e, t: (e, 0, 0)),
            pl.BlockSpec((1, 1, F), lambda e, t: (e, 0, 0)),
            pl.BlockSpec((1, F, D), lambda e, t: (e, 0, 0)),
            pl.BlockSpec((1, 1, D), lambda e, t: (e, 0, 0)),
        ],
        out_specs=pl.BlockSpec((1, D), lambda e, t: (0, 0)),
        out_shape=jax.ShapeDtypeStruct((1, D), jnp.float32),
        scratch_shapes=[
            pltpu.VMEM((E, S), jnp.float32),
            pltpu.VMEM((1, F), jnp.float32),
            pltpu.VMEM((1, D), jnp.float32),
            pltpu.SMEM((1,), jnp.float32),
            pltpu.SMEM((1,), jnp.float32),
        ],
        compiler_params=pltpu.CompilerParams(
            dimension_semantics=("arbitrary", "arbitrary"),
        ),
        interpret=_INTERPRET,
    )(x, Wg, bg.reshape(1, E), W1, b1.reshape(E, 1, F), W2, b2.reshape(E, 1, D))
    return out.reshape(1, 1, D)


# speedup vs baseline: 2.5738x; 1.0430x over previous
import jax
import jax.numpy as jnp
from jax.experimental import pallas as pl
from jax.experimental.pallas import tpu as pltpu

B, S, D, F, E, K = 1, 2048, 1024, 2048, 8, 2
ST = 2048
T = S // ST
_INTERPRET = False


def _moe_body(x_ref, wg_ref, bg_ref, w1_ref, b1_ref, w2_ref, b2_ref,
              out_ref, wmask_ref, vacc_ref, oacc_ref, tw_ref, wsum_ref):
    e = pl.program_id(0)
    t = pl.program_id(1)
    x = x_ref[...]

    @pl.when(e == 0)
    def _router():
        scores = jnp.dot(x, wg_ref[...], preferred_element_type=jnp.float32)
        scores = scores + bg_ref[...]
        m1 = jnp.max(scores, axis=1, keepdims=True)
        i1 = jnp.argmax(scores, axis=1).astype(jnp.int32)
        col = jax.lax.broadcasted_iota(jnp.int32, scores.shape, 1)
        sel1 = col == i1[:, None]
        masked = jnp.where(sel1, -jnp.inf, scores)
        m2 = jnp.max(masked, axis=1, keepdims=True)
        i2 = jnp.argmax(masked, axis=1).astype(jnp.int32)
        sel2 = col == i2[:, None]
        e2 = jnp.exp(m2 - m1)
        denom = 1.0 + e2
        wm = jnp.where(sel1, 1.0 / denom, 0.0) + jnp.where(sel2, e2 / denom, 0.0)
        wmask_ref[:, pl.ds(t * ST, ST)] = wm.T
        @pl.when(t == 0)
        def _():
            tw_ref[0] = 0.0
        tw_ref[0] += jnp.sum(wm)

    @pl.when(t == 0)
    def _():
        vacc_ref[...] = jnp.zeros_like(vacc_ref)
        wsum_ref[0] = 0.0

    w_row = wmask_ref[pl.ds(e, 1), pl.ds(t * ST, ST)]
    h = jnp.dot(x, w1_ref[0], preferred_element_type=jnp.float32)
    h = jnp.maximum(h + b1_ref[0], 0.0)
    vacc_ref[...] += jnp.dot(w_row, h, preferred_element_type=jnp.float32)
    wsum_ref[0] += jnp.sum(w_row)

    @pl.when(t == T - 1)
    def _finish_expert():
        contrib = jnp.dot(vacc_ref[...], w2_ref[0],
                          preferred_element_type=jnp.float32)
        contrib = contrib + wsum_ref[0] * b2_ref[0]
        @pl.when(e == 0)
        def _():
            oacc_ref[...] = jnp.zeros_like(oacc_ref)
        oacc_ref[...] += contrib
        @pl.when(e == E - 1)
        def _():
            out_ref[...] = oacc_ref[...] / tw_ref[0]


def kernel(input_tensor, Wg, bg, W1, b1, W2, b2):
    x = input_tensor.reshape(S, D)
    out = pl.pallas_call(
        _moe_body,
        grid=(E, T),
        in_specs=[
            pl.BlockSpec((ST, D), lambda e, t: (t, 0)),
            pl.BlockSpec((D, E), lambda e, t: (0, 0)),
            pl.BlockSpec((1, E), lambda e, t: (0, 0)),
            pl.BlockSpec((1, D, F), lambda e, t: (e, 0, 0)),
            pl.BlockSpec((1, 1, F), lambda e, t: (e, 0, 0)),
            pl.BlockSpec((1, F, D), lambda e, t: (e, 0, 0)),
            pl.BlockSpec((1, 1, D), lambda e, t: (e, 0, 0)),
        ],
        out_specs=pl.BlockSpec((1, D), lambda e, t: (0, 0)),
        out_shape=jax.ShapeDtypeStruct((1, D), jnp.float32),
        scratch_shapes=[
            pltpu.VMEM((E, S), jnp.float32),
            pltpu.VMEM((1, F), jnp.float32),
            pltpu.VMEM((1, D), jnp.float32),
            pltpu.SMEM((1,), jnp.float32),
            pltpu.SMEM((1,), jnp.float32),
        ],
        compiler_params=pltpu.CompilerParams(
            dimension_semantics=("arbitrary", "arbitrary"),
        ),
        interpret=_INTERPRET,
    )(x, Wg, bg.reshape(1, E), W1, b1.reshape(E, 1, F), W2, b2.reshape(E, 1, D))
    return out.reshape(1, 1, D)
